# baseline (speedup 1.0000x reference)
import jax
import jax.numpy as jnp
from jax import lax
from jax.experimental import pallas as pl
from jax.experimental.pallas import tpu as pltpu
from jax.experimental.pallas import tpu_sc as plsc

B = 16
D = 512
MAXL = 4096
LANES = 16
CH = 64
ZR = 32
NC = 2
NS = 16
NW = NC * NS
NWIN = MAXL // CH // NW
DV = D // LANES


def _sc_body(x_hbm, pe_hbm, len_hbm, mla_hbm, lane_hbm, out_hbm,
             len_sm, mla_sm, starts_sm, lane_v, pe_v, xb0, xb1, zb,
             gsem0, gsem1, osem0, osem1):
    wid = lax.axis_index("s") * NC + lax.axis_index("c")
    xbs = (xb0, xb1)
    gsems = (gsem0, gsem1)
    osems = (osem0, osem1)

    pltpu.sync_copy(len_hbm, len_sm.at[pl.ds(0, LANES)])
    pltpu.sync_copy(mla_hbm, mla_sm)
    pltpu.sync_copy(lane_hbm, lane_v)

    def len_at(i):
        return len_sm[pl.ds(i, LANES)][0]

    def starts_body(i, acc):
        starts_sm[i] = acc
        return acc + len_at(i)

    lax.fori_loop(0, B, starts_body, jnp.int32(0))
    mla0 = mla_sm[...][0]
    lane = lane_v[...]

    zero16 = jnp.zeros((LANES,), jnp.float32)

    def zero_row(r, _):
        for j in range(DV):
            zb[r, pl.ds(j * LANES, LANES)] = zero16
        return _

    lax.fori_loop(0, ZR, zero_row, None)

    def vparts(b, p0):
        len_b = jnp.minimum(len_at(b), mla0)
        start_b = starts_sm[b]
        return len_b - p0, start_b

    def fire_gathers(b, p0, par):
        v, start_b = vparts(b, p0)
        srow = start_b + p0
        lin = (v >= CH) & ((srow & 7) == 0)

        @pl.when(lin)
        def _():
            pltpu.async_copy(
                x_hbm.at[pl.ds(pl.multiple_of(srow, 8), CH)],
                xbs[par], gsems[par])

        @pl.when((v > 0) & jnp.logical_not(lin))
        def _():
            for i in range(CH // LANES):
                p_vec = p0 + i * LANES + lane
                src = jnp.where(p_vec < v + p0, start_b + p_vec, 0)
                pltpu.async_copy(
                    x_hbm.at[src], xbs[par].at[pl.ds(i * LANES, LANES)],
                    gsems[par])

    def drain_scatters(par):
        pltpu.make_async_copy(
            x_hbm.at[pl.ds(0, CH)], xbs[par], osems[par]).wait()

    def compute_scatter(b, p0, par):
        v, _ = vparts(b, p0)
        xb = xbs[par]
        row0 = b * MAXL + p0

        @pl.when(v > 0)
        def _():
            pltpu.make_async_copy(
                x_hbm.at[pl.ds(0, CH)], xb, gsems[par]).wait()

        @pl.when(v >= CH)
        def _():
            @plsc.parallel_loop(0, CH, 1, unroll=2)
            def _(r):
                for j in range(DV):
                    sl = pl.ds(j * LANES, LANES)
                    plsc.addupdate(xb.at[r, sl], pe_v[r, sl])

        @pl.when((v > 0) & (v < CH))
        def _():
            @plsc.parallel_loop(0, CH, 1)
            def _(r):
                m = jnp.where(r < v, 1.0, 0.0)
                m16 = jnp.full((LANES,), m, jnp.float32)
                for j in range(DV):
                    sl = pl.ds(j * LANES, LANES)
                    xb[r, sl] = (xb[r, sl] + pe_v[r, sl]) * m16

        for piece in range(CH // ZR):
            dst = out_hbm.at[
                pl.ds(pl.multiple_of(row0 + piece * ZR, ZR), ZR)]

            @pl.when(v > 0)
            def _(piece=piece, dst=dst):
                pltpu.async_copy(
                    xb.at[pl.ds(piece * ZR, ZR)], dst, osems[par])

            @pl.when(v <= 0)
            def _(dst=dst):
                pltpu.async_copy(zb, dst, osems[par])

    for win in range(NWIN):
        p0 = pl.multiple_of((wid * NWIN + win) * CH, CH)
        pltpu.sync_copy(pe_hbm.at[pl.ds(p0, CH)], pe_v)

        fire_gathers(0, p0, 0)

        def pipe_body(t, _, p0=p0):
            for P in (0, 1):
                @pl.when((t & 1) == P)
                def _(P=P):
                    @pl.when(t >= 2)
                    def _():
                        drain_scatters(P)

                    fire_gathers(t, p0, P)
                    compute_scatter(t - 1, p0, 1 - P)
            return _

        lax.fori_loop(1, B, pipe_body, None)
        compute_scatter(B - 1, p0, (B - 1) & 1)
        drain_scatters(0)
        drain_scatters(1)


@jax.jit
def kernel(x, pe, length, max_len):
    pe2 = pe.reshape(pe.shape[0], pe.shape[2])
    mla = jnp.full((LANES,), max_len, dtype=jnp.int32)
    out_flat = pl.kernel(
        _sc_body,
        out_type=jax.ShapeDtypeStruct((B * MAXL, D), jnp.float32),
        mesh=plsc.VectorSubcoreMesh(core_axis_name="c", subcore_axis_name="s"),
        scratch_types=[
            pltpu.VMEM((2 * LANES,), jnp.int32),
            pltpu.VMEM((LANES,), jnp.int32),
            pltpu.SMEM((LANES,), jnp.int32),
            pltpu.VMEM((LANES,), jnp.int32),
            pltpu.VMEM((CH, D), jnp.float32),
            pltpu.VMEM((CH, D), jnp.float32),
            pltpu.VMEM((CH, D), jnp.float32),
            pltpu.VMEM((ZR, D), jnp.float32),
            pltpu.SemaphoreType.DMA,
            pltpu.SemaphoreType.DMA,
            pltpu.SemaphoreType.DMA,
            pltpu.SemaphoreType.DMA,
        ],
    )(x, pe2, length.astype(jnp.int32), mla,
      jnp.arange(LANES, dtype=jnp.int32))
    return out_flat.reshape(B, MAXL, D)


# speedup vs baseline: 1.0725x; 1.0725x over previous
import jax
import jax.numpy as jnp
from jax import lax
from jax.experimental import pallas as pl
from jax.experimental.pallas import tpu as pltpu
from jax.experimental.pallas import tpu_sc as plsc

B = 16
D = 512
MAXL = 4096
LANES = 16
CH = 64
ZR = 32
NC = 2
NS = 16
NW = NC * NS
NWIN = MAXL // CH // NW
DV = D // LANES


def _sc_body(x_hbm, pe_hbm, len_hbm, mla_hbm, lane_hbm, out_hbm,
             len_sm, mla_sm, starts_sm, lane_v, pe_v, xb0, xb1, zb,
             gsem0, gsem1, osem0, osem1):
    wid = lax.axis_index("s") * NC + lax.axis_index("c")
    xbs = (xb0, xb1)
    gsems = (gsem0, gsem1)
    osems = (osem0, osem1)

    pltpu.sync_copy(len_hbm, len_sm.at[pl.ds(0, LANES)])
    pltpu.sync_copy(mla_hbm, mla_sm)
    pltpu.sync_copy(lane_hbm, lane_v)

    def len_at(i):
        return len_sm[pl.ds(i, LANES)][0]

    def starts_body(i, acc):
        starts_sm[i] = acc
        return acc + len_at(i)

    lax.fori_loop(0, B, starts_body, jnp.int32(0))
    mla0 = mla_sm[...][0]
    lane = lane_v[...]

    zero16 = jnp.zeros((LANES,), jnp.float32)

    def zero_row(r, _):
        for j in range(DV):
            zb[r, pl.ds(j * LANES, LANES)] = zero16
        return _

    lax.fori_loop(0, ZR, zero_row, None)

    def vparts(b, p0):
        len_b = jnp.minimum(len_at(b), mla0)
        start_b = starts_sm[b]
        return len_b - p0, start_b

    def fire_gathers(b, p0, par):
        v, start_b = vparts(b, p0)
        srow = start_b + p0
        lin = (v >= CH) & ((srow & 7) == 0)

        @pl.when(lin)
        def _():
            pltpu.async_copy(
                x_hbm.at[pl.ds(pl.multiple_of(srow, 8), CH)],
                xbs[par], gsems[par])

        @pl.when((v > 0) & jnp.logical_not(lin))
        def _():
            for i in range(CH // LANES):
                p_vec = p0 + i * LANES + lane
                src = jnp.where(p_vec < v + p0, start_b + p_vec, 0)
                pltpu.async_copy(
                    x_hbm.at[src], xbs[par].at[pl.ds(i * LANES, LANES)],
                    gsems[par])

    def drain_scatters(par):
        pltpu.make_async_copy(
            x_hbm.at[pl.ds(0, CH)], xbs[par], osems[par]).wait()

    def compute_scatter(b, p0, par):
        v, _ = vparts(b, p0)
        xb = xbs[par]
        row0 = b * MAXL + p0

        @pl.when(v > 0)
        def _():
            pltpu.make_async_copy(
                x_hbm.at[pl.ds(0, CH)], xb, gsems[par]).wait()

        @pl.when(v >= CH)
        def _():
            @plsc.parallel_loop(0, CH, 1, unroll=2)
            def _(r):
                for j in range(DV):
                    sl = pl.ds(j * LANES, LANES)
                    plsc.addupdate(xb.at[r, sl], pe_v[r, sl])

        @pl.when((v > 0) & (v < CH))
        def _():
            @plsc.parallel_loop(0, CH, 1)
            def _(r):
                m = jnp.where(r < v, 1.0, 0.0)
                m16 = jnp.full((LANES,), m, jnp.float32)
                for j in range(DV):
                    sl = pl.ds(j * LANES, LANES)
                    xb[r, sl] = (xb[r, sl] + pe_v[r, sl]) * m16

        @pl.when(v > 0)
        def _():
            pltpu.async_copy(
                xb, out_hbm.at[pl.ds(pl.multiple_of(row0, ZR), CH)],
                osems[par])

        @pl.when(v <= 0)
        def _():
            for piece in range(CH // ZR):
                pltpu.async_copy(
                    zb,
                    out_hbm.at[
                        pl.ds(pl.multiple_of(row0 + piece * ZR, ZR), ZR)],
                    osems[par])

    for win in range(NWIN):
        p0 = pl.multiple_of((wid * NWIN + win) * CH, CH)
        pltpu.sync_copy(pe_hbm.at[pl.ds(p0, CH), 0], pe_v)

        fire_gathers(0, p0, 0)

        def pipe_body(t, _, p0=p0):
            for P in (0, 1):
                @pl.when((t & 1) == P)
                def _(P=P):
                    @pl.when(t >= 2)
                    def _():
                        drain_scatters(P)

                    fire_gathers(t, p0, P)
                    compute_scatter(t - 1, p0, 1 - P)
            return _

        lax.fori_loop(1, B, pipe_body, None)
        compute_scatter(B - 1, p0, (B - 1) & 1)
        drain_scatters(0)
        drain_scatters(1)


@jax.jit
def kernel(x, pe, length, max_len):
    mla = jnp.full((LANES,), max_len, dtype=jnp.int32)
    out_flat = pl.kernel(
        _sc_body,
        out_type=jax.ShapeDtypeStruct((B * MAXL, D), jnp.float32),
        mesh=plsc.VectorSubcoreMesh(core_axis_name="c", subcore_axis_name="s"),
        scratch_types=[
            pltpu.VMEM((2 * LANES,), jnp.int32),
            pltpu.VMEM((LANES,), jnp.int32),
            pltpu.SMEM((LANES,), jnp.int32),
            pltpu.VMEM((LANES,), jnp.int32),
            pltpu.VMEM((CH, D), jnp.float32),
            pltpu.VMEM((CH, D), jnp.float32),
            pltpu.VMEM((CH, D), jnp.float32),
            pltpu.VMEM((ZR, D), jnp.float32),
            pltpu.SemaphoreType.DMA,
            pltpu.SemaphoreType.DMA,
            pltpu.SemaphoreType.DMA,
            pltpu.SemaphoreType.DMA,
        ],
    )(x, pe, length.astype(jnp.int32), mla,
      jnp.arange(LANES, dtype=jnp.int32))
    return out_flat.reshape(B, MAXL, D)


# speedup vs baseline: 1.0862x; 1.0128x over previous
import jax
import jax.numpy as jnp
from jax import lax
from jax.experimental import pallas as pl
from jax.experimental.pallas import tpu as pltpu
from jax.experimental.pallas import tpu_sc as plsc

B = 16
D = 512
MAXL = 4096
LANES = 16
CH = 32
WINR = 64
NBUF = 4
NC = 2
NS = 16
NW = NC * NS
NWIN = MAXL // WINR // NW
CPW = WINR // CH
NIT = B * CPW
DV = D // LANES


def _sc_body(x_hbm, pe_hbm, len_hbm, mla_hbm, lane_hbm, out_hbm,
             len_sm, mla_sm, starts_sm, lane_v, pe_v,
             xb0, xb1, xb2, xb3,
             gsem0, gsem1, gsem2, gsem3, osem0, osem1, osem2, osem3):
    wid = lax.axis_index("s") * NC + lax.axis_index("c")
    xbs = (xb0, xb1, xb2, xb3)
    gsems = (gsem0, gsem1, gsem2, gsem3)
    osems = (osem0, osem1, osem2, osem3)

    pltpu.sync_copy(len_hbm, len_sm.at[pl.ds(0, LANES)])
    pltpu.sync_copy(mla_hbm, mla_sm)
    pltpu.sync_copy(lane_hbm, lane_v)

    def len_at(i):
        return len_sm[pl.ds(i, LANES)][0]

    def starts_body(i, acc):
        starts_sm[i] = acc
        return acc + len_at(i)

    lax.fori_loop(0, B, starts_body, jnp.int32(0))
    mla0 = mla_sm[...][0]
    lane = lane_v[...]
    zero16 = jnp.zeros((LANES,), jnp.float32)

    def vparts(k, p0):
        b = k >> 1
        pc = p0 + (k & 1) * CH
        len_b = jnp.minimum(len_at(b), mla0)
        return len_b - pc, starts_sm[b], b, pc

    def fire_gathers(k, p0, par):
        v, start_b, _, pc = vparts(k, p0)
        srow = start_b + pc
        lin = (v >= CH) & ((srow & 7) == 0)

        @pl.when(lin)
        def _():
            pltpu.async_copy(
                x_hbm.at[pl.ds(pl.multiple_of(srow, 8), CH)],
                xbs[par], gsems[par])

        @pl.when((v > 0) & jnp.logical_not(lin))
        def _():
            for i in range(CH // LANES):
                p_vec = pc + i * LANES + lane
                src = jnp.where(p_vec < v + pc, start_b + p_vec, 0)
                pltpu.async_copy(
                    x_hbm.at[src], xbs[par].at[pl.ds(i * LANES, LANES)],
                    gsems[par])

    def drain_scatters(par):
        pltpu.make_async_copy(
            x_hbm.at[pl.ds(0, CH)], xbs[par], osems[par]).wait()

    def compute_scatter(k, p0, par):
        v, _, b, pc = vparts(k, p0)
        xb = xbs[par]
        row0 = b * MAXL + pc
        po = pc - p0

        @pl.when(v > 0)
        def _():
            pltpu.make_async_copy(
                x_hbm.at[pl.ds(0, CH)], xb, gsems[par]).wait()

        @pl.when(v >= CH)
        def _():
            @plsc.parallel_loop(0, CH, 1, unroll=2)
            def _(r):
                for j in range(DV):
                    sl = pl.ds(j * LANES, LANES)
                    plsc.addupdate(xb.at[r, sl], pe_v[po + r, sl])

        @pl.when((v > 0) & (v < CH))
        def _():
            @plsc.parallel_loop(0, CH, 1)
            def _(r):
                m = jnp.where(r < v, 1.0, 0.0)
                m16 = jnp.full((LANES,), m, jnp.float32)
                for j in range(DV):
                    sl = pl.ds(j * LANES, LANES)
                    xb[r, sl] = (xb[r, sl] + pe_v[po + r, sl]) * m16

        @pl.when(v <= 0)
        def _():
            @plsc.parallel_loop(0, CH, 1)
            def _(r):
                for j in range(DV):
                    xb[r, pl.ds(j * LANES, LANES)] = zero16

        pltpu.async_copy(
            xb, out_hbm.at[pl.ds(pl.multiple_of(row0, CH), CH)], osems[par])

    def win_body(win, _):
        p0 = pl.multiple_of((wid * NWIN + win) * WINR, WINR)
        pltpu.sync_copy(pe_hbm.at[pl.ds(p0, WINR), 0], pe_v)

        fire_gathers(0, p0, 0)

        def pipe_body(t, _, p0=p0):
            for P in range(NBUF):
                @pl.when((t & 3) == P)
                def _(P=P):
                    @pl.when(t >= NBUF)
                    def _():
                        drain_scatters(P)

                    fire_gathers(t, p0, P)
                    compute_scatter(t - 1, p0, (P - 1) % NBUF)
            return _

        lax.fori_loop(1, NIT, pipe_body, None)
        compute_scatter(NIT - 1, p0, (NIT - 1) % NBUF)
        for P in range(NBUF):
            drain_scatters(P)
        return _

    lax.fori_loop(0, NWIN, win_body, None)


@jax.jit
def kernel(x, pe, length, max_len):
    mla = jnp.full((LANES,), max_len, dtype=jnp.int32)
    out_flat = pl.kernel(
        _sc_body,
        out_type=jax.ShapeDtypeStruct((B * MAXL, D), jnp.float32),
        mesh=plsc.VectorSubcoreMesh(core_axis_name="c", subcore_axis_name="s"),
        scratch_types=(
            [pltpu.VMEM((2 * LANES,), jnp.int32),
             pltpu.VMEM((LANES,), jnp.int32),
             pltpu.SMEM((LANES,), jnp.int32),
             pltpu.VMEM((LANES,), jnp.int32),
             pltpu.VMEM((WINR, D), jnp.float32)]
            + [pltpu.VMEM((CH, D), jnp.float32)] * NBUF
            + [pltpu.SemaphoreType.DMA] * (2 * NBUF)
        ),
    )(x, pe, length.astype(jnp.int32), mla,
      jnp.arange(LANES, dtype=jnp.int32))
    return out_flat.reshape(B, MAXL, D)


# speedup vs baseline: 1.1425x; 1.0518x over previous
import jax
import jax.numpy as jnp
from jax import lax
from jax.experimental import pallas as pl
from jax.experimental.pallas import tpu as pltpu
from jax.experimental.pallas import tpu_sc as plsc

B = 16
D = 512
MAXL = 4096
LANES = 16
CH = 32
WINR = 128
NBUF = 3
NC = 2
NS = 16
NW = NC * NS
NWIN = MAXL // WINR // NW
CPW = WINR // CH
NIT = B * CPW
DV = D // LANES


def _sc_body(x_hbm, pe_hbm, len_hbm, mla_hbm, lane_hbm, out_hbm,
             len_sm, mla_sm, starts_sm, lane_v, pe_v,
             xb0, xb1, xb2,
             gsem0, gsem1, gsem2, osem0, osem1, osem2):
    wid = lax.axis_index("s") * NC + lax.axis_index("c")
    xbs = (xb0, xb1, xb2)
    gsems = (gsem0, gsem1, gsem2)
    osems = (osem0, osem1, osem2)

    pltpu.sync_copy(len_hbm, len_sm.at[pl.ds(0, LANES)])
    pltpu.sync_copy(mla_hbm, mla_sm)
    pltpu.sync_copy(lane_hbm, lane_v)

    def len_at(i):
        return len_sm[pl.ds(i, LANES)][0]

    def starts_body(i, acc):
        starts_sm[i] = acc
        return acc + len_at(i)

    lax.fori_loop(0, B, starts_body, jnp.int32(0))
    mla0 = mla_sm[...][0]
    lane = lane_v[...]
    zero16 = jnp.zeros((LANES,), jnp.float32)

    def vparts(k, p0):
        b = k // CPW
        pc = p0 + (k % CPW) * CH
        len_b = jnp.minimum(len_at(b), mla0)
        return len_b - pc, starts_sm[b], b, pc

    def fire_gathers(k, p0, par):
        v, start_b, _, pc = vparts(k, p0)
        srow = start_b + pc
        lin = (v >= CH) & ((srow & 7) == 0)

        @pl.when(lin)
        def _():
            pltpu.async_copy(
                x_hbm.at[pl.ds(pl.multiple_of(srow, 8), CH)],
                xbs[par], gsems[par])

        @pl.when((v > 0) & jnp.logical_not(lin))
        def _():
            for i in range(CH // LANES):
                p_vec = pc + i * LANES + lane
                src = jnp.where(p_vec < v + pc, start_b + p_vec, 0)
                pltpu.async_copy(
                    x_hbm.at[src], xbs[par].at[pl.ds(i * LANES, LANES)],
                    gsems[par])

    def drain_scatters(par):
        pltpu.make_async_copy(
            x_hbm.at[pl.ds(0, CH)], xbs[par], osems[par]).wait()

    def compute_scatter(k, p0, par):
        v, _, b, pc = vparts(k, p0)
        xb = xbs[par]
        row0 = b * MAXL + pc
        po = pc - p0

        @pl.when(v > 0)
        def _():
            pltpu.make_async_copy(
                x_hbm.at[pl.ds(0, CH)], xb, gsems[par]).wait()

        @pl.when(v >= CH)
        def _():
            @plsc.parallel_loop(0, CH, 1, unroll=2)
            def _(r):
                for j in range(DV):
                    sl = pl.ds(j * LANES, LANES)
                    plsc.addupdate(xb.at[r, sl], pe_v[po + r, sl])

        @pl.when((v > 0) & (v < CH))
        def _():
            @plsc.parallel_loop(0, CH, 1)
            def _(r):
                m = jnp.where(r < v, 1.0, 0.0)
                m16 = jnp.full((LANES,), m, jnp.float32)
                for j in range(DV):
                    sl = pl.ds(j * LANES, LANES)
                    xb[r, sl] = (xb[r, sl] + pe_v[po + r, sl]) * m16

        @pl.when(v <= 0)
        def _():
            @plsc.parallel_loop(0, CH, 1)
            def _(r):
                for j in range(DV):
                    xb[r, pl.ds(j * LANES, LANES)] = zero16

        pltpu.async_copy(
            xb, out_hbm.at[pl.ds(pl.multiple_of(row0, CH), CH)], osems[par])

    def win_body(win, _):
        p0 = pl.multiple_of((wid * NWIN + win) * WINR, WINR)
        pltpu.sync_copy(pe_hbm.at[pl.ds(p0, WINR), 0], pe_v)

        fire_gathers(0, p0, 0)

        def pipe_body(t, _, p0=p0):
            for P in range(NBUF):
                @pl.when(t % NBUF == P)
                def _(P=P):
                    @pl.when(t >= NBUF)
                    def _():
                        drain_scatters(P)

                    fire_gathers(t, p0, P)
                    compute_scatter(t - 1, p0, (P - 1) % NBUF)
            return _

        lax.fori_loop(1, NIT, pipe_body, None)
        compute_scatter(NIT - 1, p0, (NIT - 1) % NBUF)
        for P in range(NBUF):
            drain_scatters(P)
        return _

    lax.fori_loop(0, NWIN, win_body, None)


@jax.jit
def kernel(x, pe, length, max_len):
    mla = jnp.full((LANES,), max_len, dtype=jnp.int32)
    out_flat = pl.kernel(
        _sc_body,
        out_type=jax.ShapeDtypeStruct((B * MAXL, D), jnp.float32),
        mesh=plsc.VectorSubcoreMesh(core_axis_name="c", subcore_axis_name="s"),
        scratch_types=(
            [pltpu.VMEM((2 * LANES,), jnp.int32),
             pltpu.VMEM((LANES,), jnp.int32),
             pltpu.SMEM((LANES,), jnp.int32),
             pltpu.VMEM((LANES,), jnp.int32),
             pltpu.VMEM((WINR, D), jnp.float32)]
            + [pltpu.VMEM((CH, D), jnp.float32)] * NBUF
            + [pltpu.SemaphoreType.DMA] * (2 * NBUF)
        ),
    )(x, pe, length.astype(jnp.int32), mla,
      jnp.arange(LANES, dtype=jnp.int32))
    return out_flat.reshape(B, MAXL, D)
